# baseline (speedup 1.0000x reference)
import jax
import jax.numpy as jnp
from jax.experimental import pallas as pl
from jax.experimental.pallas import tpu as pltpu

_N = 10000
_IN = 128
_OUT = 128
_TM = 400


def _split_bf16(v):
    hi = v.astype(jnp.bfloat16)
    lo = (v - hi.astype(jnp.float32)).astype(jnp.bfloat16)
    return hi, lo


def _xw_kernel(x_ref, w_ref, hh_ref, hl_ref):
    xh, xl = _split_bf16(x_ref[...])
    wh, wl = _split_bf16(w_ref[...])
    h = jnp.dot(xh, wh, preferred_element_type=jnp.float32)
    h += jnp.dot(xh, wl, preferred_element_type=jnp.float32)
    h += jnp.dot(xl, wh, preferred_element_type=jnp.float32)
    hh, hl = _split_bf16(h)
    hh_ref[...] = hh
    hl_ref[...] = hl


def _adj_h_kernel(adj_ref, hh_ref, hl_ref, out_ref):
    ah, al = _split_bf16(adj_ref[...])
    hh = hh_ref[...]
    acc = jnp.dot(ah, hh, preferred_element_type=jnp.float32)
    acc += jnp.dot(ah, hl_ref[...], preferred_element_type=jnp.float32)
    acc += jnp.dot(al, hh, preferred_element_type=jnp.float32)
    out_ref[...] = jnp.maximum(acc, 0.0)


def kernel(x, adj, W):
    hh, hl = pl.pallas_call(
        _xw_kernel,
        out_shape=(
            jax.ShapeDtypeStruct((_N, _OUT), jnp.bfloat16),
            jax.ShapeDtypeStruct((_N, _OUT), jnp.bfloat16),
        ),
    )(x, W)

    grid = _N // _TM
    out = pl.pallas_call(
        _adj_h_kernel,
        grid=(grid,),
        in_specs=[
            pl.BlockSpec((_TM, _N), lambda i: (i, 0)),
            pl.BlockSpec((_N, _OUT), lambda i: (0, 0)),
            pl.BlockSpec((_N, _OUT), lambda i: (0, 0)),
        ],
        out_specs=pl.BlockSpec((_TM, _OUT), lambda i: (i, 0)),
        out_shape=jax.ShapeDtypeStruct((_N, _OUT), jnp.float32),
        compiler_params=pltpu.CompilerParams(
            dimension_semantics=("parallel",),
        ),
    )(adj, hh, hl)
    return out


# speedup vs baseline: 1.2610x; 1.2610x over previous
import jax
import jax.numpy as jnp
from jax.experimental import pallas as pl
from jax.experimental.pallas import tpu as pltpu

_N = 10000
_IN = 128
_OUT = 128
_TM = 400


def _split_bf16(v):
    hi = v.astype(jnp.bfloat16)
    lo = (v - hi.astype(jnp.float32)).astype(jnp.bfloat16)
    return hi, lo


def _xw_kernel(x_ref, w_ref, hh_ref, hl_ref):
    xh, xl = _split_bf16(x_ref[...])
    wh, wl = _split_bf16(w_ref[...])
    h = jnp.dot(xh, wh, preferred_element_type=jnp.float32)
    h += jnp.dot(xh, wl, preferred_element_type=jnp.float32)
    h += jnp.dot(xl, wh, preferred_element_type=jnp.float32)
    hh, hl = _split_bf16(h)
    hh_ref[...] = hh
    hl_ref[...] = hl


def _adj_h_kernel(adj_ref, hh_ref, hl_ref, out_ref):
    ah = adj_ref[...].astype(jnp.bfloat16)
    acc = jnp.dot(ah, hh_ref[...], preferred_element_type=jnp.float32)
    acc += jnp.dot(ah, hl_ref[...], preferred_element_type=jnp.float32)
    out_ref[...] = jnp.maximum(acc, 0.0)


def kernel(x, adj, W):
    hh, hl = pl.pallas_call(
        _xw_kernel,
        out_shape=(
            jax.ShapeDtypeStruct((_N, _OUT), jnp.bfloat16),
            jax.ShapeDtypeStruct((_N, _OUT), jnp.bfloat16),
        ),
    )(x, W)

    grid = _N // _TM
    out = pl.pallas_call(
        _adj_h_kernel,
        grid=(grid,),
        in_specs=[
            pl.BlockSpec((_TM, _N), lambda i: (i, 0)),
            pl.BlockSpec((_N, _OUT), lambda i: (0, 0)),
            pl.BlockSpec((_N, _OUT), lambda i: (0, 0)),
        ],
        out_specs=pl.BlockSpec((_TM, _OUT), lambda i: (i, 0)),
        out_shape=jax.ShapeDtypeStruct((_N, _OUT), jnp.float32),
        compiler_params=pltpu.CompilerParams(
            dimension_semantics=("parallel",),
        ),
    )(adj, hh, hl)
    return out


# speedup vs baseline: 1.3338x; 1.0577x over previous
import jax
import jax.numpy as jnp
from jax.experimental import pallas as pl
from jax.experimental.pallas import tpu as pltpu

_N = 10000
_IN = 128
_OUT = 128
_TM = 400


def _split_bf16(v):
    hi = v.astype(jnp.bfloat16)
    lo = (v - hi.astype(jnp.float32)).astype(jnp.bfloat16)
    return hi, lo


def _xw_kernel(x_ref, w_ref, h2_ref):
    xh, xl = _split_bf16(x_ref[...])
    wh, wl = _split_bf16(w_ref[...])
    h = jnp.dot(xh, wh, preferred_element_type=jnp.float32)
    h += jnp.dot(xh, wl, preferred_element_type=jnp.float32)
    h += jnp.dot(xl, wh, preferred_element_type=jnp.float32)
    hh, hl = _split_bf16(h)
    h2_ref[...] = jnp.concatenate([hh, hl], axis=1)


def _adj_h_kernel(adj_ref, h2_ref, out_ref):
    ah = adj_ref[...].astype(jnp.bfloat16)
    acc = jnp.dot(ah, h2_ref[...], preferred_element_type=jnp.float32)
    out_ref[...] = jnp.maximum(acc[:, :_OUT] + acc[:, _OUT:], 0.0)


def kernel(x, adj, W):
    h2 = pl.pallas_call(
        _xw_kernel,
        out_shape=jax.ShapeDtypeStruct((_N, 2 * _OUT), jnp.bfloat16),
    )(x, W)

    grid = _N // _TM
    out = pl.pallas_call(
        _adj_h_kernel,
        grid=(grid,),
        in_specs=[
            pl.BlockSpec((_TM, _N), lambda i: (i, 0)),
            pl.BlockSpec((_N, 2 * _OUT), lambda i: (0, 0)),
        ],
        out_specs=pl.BlockSpec((_TM, _OUT), lambda i: (i, 0)),
        out_shape=jax.ShapeDtypeStruct((_N, _OUT), jnp.float32),
        compiler_params=pltpu.CompilerParams(
            dimension_semantics=("parallel",),
        ),
    )(adj, h2)
    return out


# speedup vs baseline: 1.3987x; 1.0486x over previous
import jax
import jax.numpy as jnp
from jax.experimental import pallas as pl
from jax.experimental.pallas import tpu as pltpu

_N = 10000
_IN = 128
_OUT = 128
_TM = 400


def _split_bf16(v):
    hi = v.astype(jnp.bfloat16)
    lo = (v - hi.astype(jnp.float32)).astype(jnp.bfloat16)
    return hi, lo


def _fused_kernel(x_ref, w_ref, adj_ref, out_ref, h2_ref):
    @pl.when(pl.program_id(0) == 0)
    def _():
        xh, xl = _split_bf16(x_ref[...])
        wh, wl = _split_bf16(w_ref[...])
        h = jnp.dot(xh, wh, preferred_element_type=jnp.float32)
        h += jnp.dot(xh, wl, preferred_element_type=jnp.float32)
        h += jnp.dot(xl, wh, preferred_element_type=jnp.float32)
        hh, hl = _split_bf16(h)
        h2_ref[...] = jnp.concatenate([hh, hl], axis=1)

    ah = adj_ref[...].astype(jnp.bfloat16)
    acc = jnp.dot(ah, h2_ref[...], preferred_element_type=jnp.float32)
    out_ref[...] = jnp.maximum(acc[:, :_OUT] + acc[:, _OUT:], 0.0)


def kernel(x, adj, W):
    grid = _N // _TM
    return pl.pallas_call(
        _fused_kernel,
        grid=(grid,),
        in_specs=[
            pl.BlockSpec((_N, _IN), lambda i: (0, 0)),
            pl.BlockSpec((_IN, _OUT), lambda i: (0, 0)),
            pl.BlockSpec((_TM, _N), lambda i: (i, 0)),
        ],
        out_specs=pl.BlockSpec((_TM, _OUT), lambda i: (i, 0)),
        out_shape=jax.ShapeDtypeStruct((_N, _OUT), jnp.float32),
        scratch_shapes=[pltpu.VMEM((_N, 2 * _OUT), jnp.bfloat16)],
        compiler_params=pltpu.CompilerParams(
            dimension_semantics=("arbitrary",),
        ),
    )(x, W, adj)
